# Initial kernel scaffold; baseline (speedup 1.0000x reference)
import jax
import jax.numpy as jnp
from jax.experimental import pallas as pl


def kernel(x, W, root_weight, bias, edge_index, edge_type):
    raise NotImplementedError("write your pallas kernel here")


# speedup vs baseline: 10.8182x; 10.8182x over previous
import dataclasses
import functools

import jax
import jax.numpy as jnp
from jax import lax
from jax.experimental import pallas as pl
from jax.experimental.pallas import tpu as pltpu
from jax.experimental.pallas import tpu_sc as plsc

N = 10000
E = 320000
D = 128
R = 8

NS = 16
BLK = 80
NBLK = E // BLK
TPB = NBLK // NS
NSB = 8
NGRP = TPB // NSB
TAIL = TPB - NGRP * NSB
ZCH = 80
NZC = N // ZCH
DEGSZ = N * R
DZCH = 1600
NDZC = DEGSZ // DZCH
MB = 1000


def _mm_body(x_ref, w_ref, h_ref):
    h_ref[...] = lax.dot_general(
        x_ref[...], w_ref[0],
        dimension_numbers=(((1,), (0,)), ((), ())),
        preferred_element_type=jnp.float32)


def _relation_matmuls(x, wall):
    nt = N // MB
    return pl.pallas_call(
        _mm_body,
        grid=(R + 1, nt),
        in_specs=[
            pl.BlockSpec((MB, D), lambda r, i: (i, 0)),
            pl.BlockSpec((1, D, D), lambda r, i: (r, 0, 0)),
        ],
        out_specs=pl.BlockSpec((MB, D), lambda r, i: (r * nt + i, 0)),
        out_shape=jax.ShapeDtypeStruct(((R + 1) * N, D), jnp.float32),
    )(x, wall)


_sc_mesh = plsc.VectorSubcoreMesh(
    core_axis_name="c", subcore_axis_name="s", num_cores=1)

_sc_params = pltpu.CompilerParams()
if "needs_layout_passes" in pltpu.CompilerParams.__dataclass_fields__:
    _sc_params = dataclasses.replace(_sc_params, needs_layout_passes=False)


@functools.partial(
    pl.kernel,
    out_type=jax.ShapeDtypeStruct((N, D), jnp.float32),
    mesh=_sc_mesh,
    compiler_params=_sc_params,
    scratch_types=[
        pltpu.VMEM((NSB, BLK), jnp.int32),
        pltpu.VMEM((NSB, BLK), jnp.int32),
        pltpu.VMEM((NSB, BLK), jnp.int32),
        pltpu.VMEM((BLK, D), jnp.float32),
        pltpu.VMEM((BLK,), jnp.float32),
        pltpu.VMEM((BLK,), jnp.float32),
        pltpu.VMEM((BLK,), jnp.float32),
        pltpu.VMEM((DZCH,), jnp.float32),
        pltpu.VMEM_SHARED((DEGSZ,), jnp.float32),
        pltpu.VMEM_SHARED((N, D), jnp.float32),
    ],
)
def _sc_edges(didx_hbm, gidx_hbm, tgt_hbm, h_hbm, outp_hbm,
              didx_v, gidx_v, tgt_v, rows_v,
              degv_v, scale_v, ones_v, zeros_v, deg_sp, out_sp):
    s = lax.axis_index("s")

    @pl.loop(0, BLK, step=16)
    def _(i):
        ones_v[pl.ds(i, 16)] = jnp.full((16,), 1.0, jnp.float32)

    @pl.loop(0, DZCH, step=16)
    def _(i):
        zeros_v[pl.ds(i, 16)] = jnp.full((16,), 0.0, jnp.float32)

    @pl.loop(s, NDZC, step=NS)
    def _(k):
        pltpu.sync_copy(zeros_v, deg_sp.at[pl.ds(k * DZCH, DZCH)])

    @pl.loop(0, BLK)
    def _(i):
        for f in range(D // 16):
            rows_v[i, pl.ds(f * 16, 16)] = jnp.full((16,), 0.0, jnp.float32)

    @pl.loop(s, NZC, step=NS)
    def _(k):
        pltpu.sync_copy(rows_v, out_sp.at[pl.ds(k * ZCH, ZCH)])

    plsc.subcore_barrier()

    @pl.loop(0, NGRP)
    def _(g):
        pltpu.sync_copy(didx_hbm.at[s].at[pl.ds(g * NSB, NSB)], didx_v)

        @pl.loop(0, NSB)
        def _(j):
            pltpu.sync_copy(ones_v, deg_sp.at[didx_v.at[j]], add=True)

    pltpu.sync_copy(didx_hbm.at[s].at[pl.ds(NGRP * NSB, TAIL)],
                    didx_v.at[pl.ds(0, TAIL)])
    for j in range(TAIL):
        pltpu.sync_copy(ones_v, deg_sp.at[didx_v.at[j]], add=True)

    plsc.subcore_barrier()

    def _edge_block(j):
        pltpu.sync_copy(deg_sp.at[didx_v.at[j]], degv_v)
        pltpu.sync_copy(h_hbm.at[gidx_v.at[j]], rows_v)

        @pl.loop(0, BLK, step=16)
        def _(i):
            scale_v[pl.ds(i, 16)] = (
                1.0 / jnp.maximum(degv_v[pl.ds(i, 16)], 1.0))

        @pl.loop(0, BLK)
        def _(i):
            sv = plsc.load_gather(
                scale_v, [lax.broadcast_in_dim(i, (16,), ())])
            for f in range(D // 16):
                rows_v[i, pl.ds(f * 16, 16)] = (
                    rows_v[i, pl.ds(f * 16, 16)] * sv)

        pltpu.sync_copy(rows_v, out_sp.at[tgt_v.at[j]], add=True)

    @pl.loop(0, NGRP)
    def _(g):
        pltpu.sync_copy(didx_hbm.at[s].at[pl.ds(g * NSB, NSB)], didx_v)
        pltpu.sync_copy(gidx_hbm.at[s].at[pl.ds(g * NSB, NSB)], gidx_v)
        pltpu.sync_copy(tgt_hbm.at[s].at[pl.ds(g * NSB, NSB)], tgt_v)

        @pl.loop(0, NSB)
        def _(j):
            _edge_block(j)

    pltpu.sync_copy(didx_hbm.at[s].at[pl.ds(NGRP * NSB, TAIL)],
                    didx_v.at[pl.ds(0, TAIL)])
    pltpu.sync_copy(gidx_hbm.at[s].at[pl.ds(NGRP * NSB, TAIL)],
                    gidx_v.at[pl.ds(0, TAIL)])
    pltpu.sync_copy(tgt_hbm.at[s].at[pl.ds(NGRP * NSB, TAIL)],
                    tgt_v.at[pl.ds(0, TAIL)])
    for j in range(TAIL):
        _edge_block(j)

    plsc.subcore_barrier()

    @pl.loop(s, NZC, step=NS)
    def _(k):
        pltpu.sync_copy(out_sp.at[pl.ds(k * ZCH, ZCH)],
                        outp_hbm.at[pl.ds(k * ZCH, ZCH)])


def _comb_body(p_ref, r_ref, b_ref, o_ref):
    o_ref[...] = p_ref[...] + r_ref[...] + b_ref[...]


def _combine(outp, h, bias2d):
    nt = N // MB
    return pl.pallas_call(
        _comb_body,
        grid=(nt,),
        in_specs=[
            pl.BlockSpec((MB, D), lambda i: (i, 0)),
            pl.BlockSpec((MB, D), lambda i: (R * nt + i, 0)),
            pl.BlockSpec((1, D), lambda i: (0, 0)),
        ],
        out_specs=pl.BlockSpec((MB, D), lambda i: (i, 0)),
        out_shape=jax.ShapeDtypeStruct((N, D), jnp.float32),
    )(outp, h, bias2d)


def kernel(x, W, root_weight, bias, edge_index, edge_type):
    x = x.astype(jnp.float32)
    src = edge_index[0].astype(jnp.int32)
    tgt = edge_index[1].astype(jnp.int32)
    et = edge_type.astype(jnp.int32)

    wall = jnp.concatenate([W.astype(jnp.float32),
                            root_weight.astype(jnp.float32)[None]], axis=0)
    h = _relation_matmuls(x, wall)

    gidx = (et * N + src).reshape(NS, TPB, BLK)
    didx = (tgt * R + et).reshape(NS, TPB, BLK)
    tgt_r = tgt.reshape(NS, TPB, BLK)

    outp = _sc_edges(didx, gidx, tgt_r, h)

    return _combine(outp, h, bias.astype(jnp.float32).reshape(1, D))


# speedup vs baseline: 15.4428x; 1.4275x over previous
import dataclasses
import functools

import jax
import jax.numpy as jnp
from jax import lax
from jax.experimental import pallas as pl
from jax.experimental.pallas import tpu as pltpu
from jax.experimental.pallas import tpu_sc as plsc

N = 10000
E = 320000
D = 128
R = 8

NS = 16
BLK = 80
NBLK = E // BLK
TPB = NBLK // NS
NSB = 8
NGRP = TPB // NSB
TAIL = TPB - NGRP * NSB
ZCH = 80
NZC = N // ZCH
DEGSZ = N * R
DZCH = 1600
NDZC = DEGSZ // DZCH
MB = 1000


def _mm_body(x_ref, w_ref, h_ref):
    h_ref[...] = lax.dot_general(
        x_ref[...], w_ref[0],
        dimension_numbers=(((1,), (0,)), ((), ())),
        preferred_element_type=jnp.float32)


def _relation_matmuls(x, wall):
    nt = N // MB
    return pl.pallas_call(
        _mm_body,
        grid=(R + 1, nt),
        in_specs=[
            pl.BlockSpec((MB, D), lambda r, i: (i, 0)),
            pl.BlockSpec((1, D, D), lambda r, i: (r, 0, 0)),
        ],
        out_specs=pl.BlockSpec((MB, D), lambda r, i: (r * nt + i, 0)),
        out_shape=jax.ShapeDtypeStruct(((R + 1) * N, D), jnp.float32),
    )(x, wall)


_sc_mesh = plsc.VectorSubcoreMesh(
    core_axis_name="c", subcore_axis_name="s", num_cores=1)

_sc_params = pltpu.CompilerParams()
if "needs_layout_passes" in pltpu.CompilerParams.__dataclass_fields__:
    _sc_params = dataclasses.replace(_sc_params, needs_layout_passes=False)


@functools.partial(
    pl.kernel,
    out_type=jax.ShapeDtypeStruct((N, D), jnp.float32),
    mesh=_sc_mesh,
    compiler_params=_sc_params,
    scratch_types=[
        pltpu.VMEM((NSB, BLK), jnp.int32),
        pltpu.VMEM((NSB, BLK), jnp.int32),
        pltpu.VMEM((NSB, BLK), jnp.int32),
        pltpu.VMEM((2, BLK, D), jnp.float32),
        pltpu.VMEM((2, BLK), jnp.float32),
        pltpu.VMEM((2, BLK), jnp.float32),
        pltpu.VMEM((BLK,), jnp.float32),
        pltpu.VMEM((DZCH,), jnp.float32),
        pltpu.VMEM_SHARED((DEGSZ,), jnp.float32),
        pltpu.VMEM_SHARED((N, D), jnp.float32),
        pltpu.SemaphoreType.DMA,
        pltpu.SemaphoreType.DMA,
        pltpu.SemaphoreType.DMA,
        pltpu.SemaphoreType.DMA,
        pltpu.SemaphoreType.DMA,
        pltpu.SemaphoreType.DMA,
    ],
)
def _sc_edges(didx_hbm, gidx_hbm, tgt_hbm, h_hbm, outp_hbm,
              didx_v, gidx_v, tgt_v, rows_v,
              degv_v, scale_v, ones_v, zeros_v, deg_sp, out_sp,
              sem_g0, sem_g1, sem_d0, sem_d1, sem_s0, sem_s1):
    s = lax.axis_index("s")

    @pl.loop(0, BLK, step=16)
    def _(i):
        ones_v[pl.ds(i, 16)] = jnp.full((16,), 1.0, jnp.float32)

    @pl.loop(0, DZCH, step=16)
    def _(i):
        zeros_v[pl.ds(i, 16)] = jnp.full((16,), 0.0, jnp.float32)

    @pl.loop(s, NDZC, step=NS)
    def _(k):
        pltpu.sync_copy(zeros_v, deg_sp.at[pl.ds(k * DZCH, DZCH)])

    @pl.loop(0, BLK)
    def _(i):
        for f in range(D // 16):
            rows_v[0, i, pl.ds(f * 16, 16)] = (
                jnp.full((16,), 0.0, jnp.float32))

    @pl.loop(s, NZC, step=NS)
    def _(k):
        pltpu.sync_copy(rows_v.at[0], out_sp.at[pl.ds(k * ZCH, ZCH)])

    plsc.subcore_barrier()

    @pl.loop(0, NGRP)
    def _(g):
        pltpu.sync_copy(didx_hbm.at[s].at[pl.ds(g * NSB, NSB)], didx_v)

        @pl.loop(0, NSB)
        def _(j):
            pltpu.sync_copy(ones_v, deg_sp.at[didx_v.at[j]], add=True)

    pltpu.sync_copy(didx_hbm.at[s].at[pl.ds(NGRP * NSB, TAIL)],
                    didx_v.at[pl.ds(0, TAIL)])
    for j in range(TAIL):
        pltpu.sync_copy(ones_v, deg_sp.at[didx_v.at[j]], add=True)

    plsc.subcore_barrier()

    sem_g = (sem_g0, sem_g1)
    sem_d = (sem_d0, sem_d1)
    sem_s = (sem_s0, sem_s1)

    def _issue(j, p):
        dg = pltpu.async_copy(deg_sp.at[didx_v.at[j]], degv_v.at[p],
                              sem_d[p])
        rg = pltpu.async_copy(h_hbm.at[gidx_v.at[j]], rows_v.at[p],
                              sem_g[p])
        return dg, rg

    def _compute(p):
        @pl.loop(0, BLK, step=16)
        def _(i):
            scale_v[p, pl.ds(i, 16)] = (
                1.0 / jnp.maximum(degv_v[p, pl.ds(i, 16)], 1.0))

        @pl.loop(0, BLK)
        def _(i):
            sv = plsc.load_gather(
                scale_v,
                [jnp.full((16,), p, jnp.int32),
                 lax.broadcast_in_dim(i, (16,), ())])
            for f in range(D // 16):
                rows_v[p, i, pl.ds(f * 16, 16)] = (
                    rows_v[p, i, pl.ds(f * 16, 16)] * sv)

    def _pipeline(js):
        gath = [None, None]
        scat = [None, None]
        gath[0] = _issue(js[0], 0)
        for k, j in enumerate(js):
            p = k & 1
            q = p ^ 1
            if k + 1 < len(js):
                if scat[q] is not None:
                    scat[q].wait()
                    scat[q] = None
                gath[q] = _issue(js[k + 1], q)
            dg, rg = gath[p]
            dg.wait()
            rg.wait()
            _compute(p)
            scat[p] = pltpu.async_copy(rows_v.at[p],
                                       out_sp.at[tgt_v.at[j]], sem_s[p],
                                       add=True)
        for p in (0, 1):
            if scat[p] is not None:
                scat[p].wait()

    @pl.loop(0, NGRP)
    def _(g):
        pltpu.sync_copy(didx_hbm.at[s].at[pl.ds(g * NSB, NSB)], didx_v)
        pltpu.sync_copy(gidx_hbm.at[s].at[pl.ds(g * NSB, NSB)], gidx_v)
        pltpu.sync_copy(tgt_hbm.at[s].at[pl.ds(g * NSB, NSB)], tgt_v)
        _pipeline(list(range(NSB)))

    pltpu.sync_copy(didx_hbm.at[s].at[pl.ds(NGRP * NSB, TAIL)],
                    didx_v.at[pl.ds(0, TAIL)])
    pltpu.sync_copy(gidx_hbm.at[s].at[pl.ds(NGRP * NSB, TAIL)],
                    gidx_v.at[pl.ds(0, TAIL)])
    pltpu.sync_copy(tgt_hbm.at[s].at[pl.ds(NGRP * NSB, TAIL)],
                    tgt_v.at[pl.ds(0, TAIL)])
    _pipeline(list(range(TAIL)))

    plsc.subcore_barrier()

    @pl.loop(s, NZC, step=NS)
    def _(k):
        pltpu.sync_copy(out_sp.at[pl.ds(k * ZCH, ZCH)],
                        outp_hbm.at[pl.ds(k * ZCH, ZCH)])


def _comb_body(p_ref, r_ref, b_ref, o_ref):
    o_ref[...] = p_ref[...] + r_ref[...] + b_ref[...]


def _combine(outp, h, bias2d):
    nt = N // MB
    return pl.pallas_call(
        _comb_body,
        grid=(nt,),
        in_specs=[
            pl.BlockSpec((MB, D), lambda i: (i, 0)),
            pl.BlockSpec((MB, D), lambda i: (R * nt + i, 0)),
            pl.BlockSpec((1, D), lambda i: (0, 0)),
        ],
        out_specs=pl.BlockSpec((MB, D), lambda i: (i, 0)),
        out_shape=jax.ShapeDtypeStruct((N, D), jnp.float32),
    )(outp, h, bias2d)


def kernel(x, W, root_weight, bias, edge_index, edge_type):
    x = x.astype(jnp.float32)
    src = edge_index[0].astype(jnp.int32)
    tgt = edge_index[1].astype(jnp.int32)
    et = edge_type.astype(jnp.int32)

    wall = jnp.concatenate([W.astype(jnp.float32),
                            root_weight.astype(jnp.float32)[None]], axis=0)
    h = _relation_matmuls(x, wall)

    gidx = (et * N + src).reshape(NS, TPB, BLK)
    didx = (tgt * R + et).reshape(NS, TPB, BLK)
    tgt_r = tgt.reshape(NS, TPB, BLK)

    outp = _sc_edges(didx, gidx, tgt_r, h)

    return _combine(outp, h, bias.astype(jnp.float32).reshape(1, D))


# speedup vs baseline: 15.8622x; 1.0272x over previous
import dataclasses
import functools

import jax
import jax.numpy as jnp
from jax import lax
from jax.experimental import pallas as pl
from jax.experimental.pallas import tpu as pltpu
from jax.experimental.pallas import tpu_sc as plsc

N = 10000
E = 320000
D = 128
R = 8

NS = 16
BLK = 80
NBLK = E // BLK
TPB = NBLK // NS
NSB = 8
NGRP = TPB // NSB
TAIL = TPB - NGRP * NSB
ZCH = 80
NZC = N // ZCH
DEGSZ = N * R
DZCH = 1600
NDZC = DEGSZ // DZCH
MB = 1000


def _mm_body(x_ref, w_ref, h_ref):
    h_ref[...] = lax.dot_general(
        x_ref[...], w_ref[0],
        dimension_numbers=(((1,), (0,)), ((), ())),
        preferred_element_type=jnp.float32)


def _relation_matmuls(x, wall):
    nt = N // MB
    return pl.pallas_call(
        _mm_body,
        grid=(R + 1, nt),
        in_specs=[
            pl.BlockSpec((MB, D), lambda r, i: (i, 0)),
            pl.BlockSpec((1, D, D), lambda r, i: (r, 0, 0)),
        ],
        out_specs=pl.BlockSpec((MB, D), lambda r, i: (r * nt + i, 0)),
        out_shape=jax.ShapeDtypeStruct(((R + 1) * N, D), jnp.float32),
    )(x, wall)


_sc_mesh = plsc.VectorSubcoreMesh(
    core_axis_name="c", subcore_axis_name="s", num_cores=1)

_sc_params = pltpu.CompilerParams()
if "needs_layout_passes" in pltpu.CompilerParams.__dataclass_fields__:
    _sc_params = dataclasses.replace(_sc_params, needs_layout_passes=False)


@functools.partial(
    pl.kernel,
    out_type=jax.ShapeDtypeStruct((N, D), jnp.float32),
    mesh=_sc_mesh,
    compiler_params=_sc_params,
    scratch_types=[
        pltpu.VMEM((NSB, BLK), jnp.int32),
        pltpu.VMEM((NSB, BLK), jnp.int32),
        pltpu.VMEM((NSB, BLK), jnp.int32),
        pltpu.VMEM((2, BLK, D), jnp.float32),
        pltpu.VMEM((2, BLK), jnp.float32),
        pltpu.VMEM((2, BLK), jnp.float32),
        pltpu.VMEM((BLK,), jnp.float32),
        pltpu.VMEM((DZCH,), jnp.float32),
        pltpu.VMEM_SHARED((DEGSZ,), jnp.float32),
        pltpu.VMEM_SHARED((N, D), jnp.float32),
        pltpu.SemaphoreType.DMA,
        pltpu.SemaphoreType.DMA,
        pltpu.SemaphoreType.DMA,
        pltpu.SemaphoreType.DMA,
        pltpu.SemaphoreType.DMA,
        pltpu.SemaphoreType.DMA,
    ],
)
def _sc_edges(didx_hbm, gidx_hbm, tgt_hbm, h_hbm, outp_hbm,
              didx_v, gidx_v, tgt_v, rows_v,
              degv_v, scale_v, ones_v, zeros_v, deg_sp, out_sp,
              sem_g0, sem_g1, sem_d0, sem_d1, sem_s0, sem_s1):
    s = lax.axis_index("s")

    @pl.loop(0, BLK, step=16)
    def _(i):
        ones_v[pl.ds(i, 16)] = jnp.full((16,), 1.0, jnp.float32)

    @pl.loop(0, DZCH, step=16)
    def _(i):
        zeros_v[pl.ds(i, 16)] = jnp.full((16,), 0.0, jnp.float32)

    @pl.loop(s, NDZC, step=NS)
    def _(k):
        pltpu.sync_copy(zeros_v, deg_sp.at[pl.ds(k * DZCH, DZCH)])

    @pl.loop(0, BLK)
    def _(i):
        for f in range(D // 16):
            rows_v[0, i, pl.ds(f * 16, 16)] = (
                jnp.full((16,), 0.0, jnp.float32))

    @pl.loop(s, NZC, step=NS)
    def _(k):
        pltpu.sync_copy(rows_v.at[0], out_sp.at[pl.ds(k * ZCH, ZCH)])

    plsc.subcore_barrier()

    @pl.loop(0, NGRP)
    def _(g):
        pltpu.sync_copy(didx_hbm.at[s].at[pl.ds(g * NSB, NSB)], didx_v)

        @pl.loop(0, NSB)
        def _(j):
            pltpu.sync_copy(ones_v, deg_sp.at[didx_v.at[j]], add=True)

    pltpu.sync_copy(didx_hbm.at[s].at[pl.ds(NGRP * NSB, TAIL)],
                    didx_v.at[pl.ds(0, TAIL)])
    for j in range(TAIL):
        pltpu.sync_copy(ones_v, deg_sp.at[didx_v.at[j]], add=True)

    plsc.subcore_barrier()

    sem_g = (sem_g0, sem_g1)
    sem_d = (sem_d0, sem_d1)
    sem_s = (sem_s0, sem_s1)

    def _issue(j, p):
        dg = pltpu.async_copy(deg_sp.at[didx_v.at[j]], degv_v.at[p],
                              sem_d[p])
        rg = pltpu.async_copy(h_hbm.at[gidx_v.at[j]], rows_v.at[p],
                              sem_g[p])
        return dg, rg

    def _compute(p):
        @pl.loop(0, BLK, step=16)
        def _(i):
            scale_v[p, pl.ds(i, 16)] = (
                1.0 / jnp.maximum(degv_v[p, pl.ds(i, 16)], 1.0))

        @pl.loop(0, BLK, unroll=4)
        def _(i):
            sv = plsc.load_gather(
                scale_v,
                [jnp.full((16,), p, jnp.int32),
                 lax.broadcast_in_dim(i, (16,), ())])
            for f in range(D // 16):
                rows_v[p, i, pl.ds(f * 16, 16)] = (
                    rows_v[p, i, pl.ds(f * 16, 16)] * sv)

    def _pipeline(js):
        gath = [None, None]
        scat = [None, None]
        gath[0] = _issue(js[0], 0)
        for k, j in enumerate(js):
            p = k & 1
            q = p ^ 1
            if k + 1 < len(js):
                if scat[q] is not None:
                    scat[q].wait()
                    scat[q] = None
                gath[q] = _issue(js[k + 1], q)
            dg, rg = gath[p]
            dg.wait()
            rg.wait()
            _compute(p)
            scat[p] = pltpu.async_copy(rows_v.at[p],
                                       out_sp.at[tgt_v.at[j]], sem_s[p],
                                       add=True)
        for p in (0, 1):
            if scat[p] is not None:
                scat[p].wait()

    @pl.loop(0, NGRP)
    def _(g):
        pltpu.sync_copy(didx_hbm.at[s].at[pl.ds(g * NSB, NSB)], didx_v)
        pltpu.sync_copy(gidx_hbm.at[s].at[pl.ds(g * NSB, NSB)], gidx_v)
        pltpu.sync_copy(tgt_hbm.at[s].at[pl.ds(g * NSB, NSB)], tgt_v)
        _pipeline(list(range(NSB)))

    pltpu.sync_copy(didx_hbm.at[s].at[pl.ds(NGRP * NSB, TAIL)],
                    didx_v.at[pl.ds(0, TAIL)])
    pltpu.sync_copy(gidx_hbm.at[s].at[pl.ds(NGRP * NSB, TAIL)],
                    gidx_v.at[pl.ds(0, TAIL)])
    pltpu.sync_copy(tgt_hbm.at[s].at[pl.ds(NGRP * NSB, TAIL)],
                    tgt_v.at[pl.ds(0, TAIL)])
    _pipeline(list(range(TAIL)))

    plsc.subcore_barrier()

    @pl.loop(s, NZC, step=NS)
    def _(k):
        pltpu.sync_copy(out_sp.at[pl.ds(k * ZCH, ZCH)],
                        outp_hbm.at[pl.ds(k * ZCH, ZCH)])


def _comb_body(p_ref, r_ref, b_ref, o_ref):
    o_ref[...] = p_ref[...] + r_ref[...] + b_ref[...]


def _combine(outp, h, bias2d):
    nt = N // MB
    return pl.pallas_call(
        _comb_body,
        grid=(nt,),
        in_specs=[
            pl.BlockSpec((MB, D), lambda i: (i, 0)),
            pl.BlockSpec((MB, D), lambda i: (R * nt + i, 0)),
            pl.BlockSpec((1, D), lambda i: (0, 0)),
        ],
        out_specs=pl.BlockSpec((MB, D), lambda i: (i, 0)),
        out_shape=jax.ShapeDtypeStruct((N, D), jnp.float32),
    )(outp, h, bias2d)


def kernel(x, W, root_weight, bias, edge_index, edge_type):
    x = x.astype(jnp.float32)
    src = edge_index[0].astype(jnp.int32)
    tgt = edge_index[1].astype(jnp.int32)
    et = edge_type.astype(jnp.int32)

    wall = jnp.concatenate([W.astype(jnp.float32),
                            root_weight.astype(jnp.float32)[None]], axis=0)
    h = _relation_matmuls(x, wall)

    gidx = (et * N + src).reshape(NS, TPB, BLK)
    didx = (tgt * R + et).reshape(NS, TPB, BLK)
    tgt_r = tgt.reshape(NS, TPB, BLK)

    outp = _sc_edges(didx, gidx, tgt_r, h)

    return _combine(outp, h, bias.astype(jnp.float32).reshape(1, D))


# speedup vs baseline: 17.8066x; 1.1226x over previous
import dataclasses
import functools

import jax
import jax.numpy as jnp
from jax import lax
from jax.experimental import pallas as pl
from jax.experimental.pallas import tpu as pltpu
from jax.experimental.pallas import tpu_sc as plsc

N = 10000
E = 320000
D = 128
R = 8

NS = 16
BLK = 80
NBLK = E // BLK
TPB = NBLK // NS
NSB = 16
NGRP = TPB // NSB
TAIL = TPB - NGRP * NSB
ZCH = 80
NZC = N // ZCH
DEGSZ = N * R
DZCH = 1600
NDZC = DEGSZ // DZCH
MB = 1000


def _mm_body(x_ref, w_ref, h_ref):
    h_ref[...] = lax.dot_general(
        x_ref[...], w_ref[0],
        dimension_numbers=(((1,), (0,)), ((), ())),
        preferred_element_type=jnp.float32)


def _relation_matmuls(x, wall):
    nt = N // MB
    return pl.pallas_call(
        _mm_body,
        grid=(R + 1, nt),
        in_specs=[
            pl.BlockSpec((MB, D), lambda r, i: (i, 0)),
            pl.BlockSpec((1, D, D), lambda r, i: (r, 0, 0)),
        ],
        out_specs=pl.BlockSpec((MB, D), lambda r, i: (r * nt + i, 0)),
        out_shape=jax.ShapeDtypeStruct(((R + 1) * N, D), jnp.float32),
    )(x, wall)


_sc_mesh = plsc.VectorSubcoreMesh(
    core_axis_name="c", subcore_axis_name="s", num_cores=1)

_sc_params = pltpu.CompilerParams()
if "needs_layout_passes" in pltpu.CompilerParams.__dataclass_fields__:
    _sc_params = dataclasses.replace(_sc_params, needs_layout_passes=False)


@functools.partial(
    pl.kernel,
    out_type=jax.ShapeDtypeStruct((N, D), jnp.float32),
    mesh=_sc_mesh,
    compiler_params=_sc_params,
    scratch_types=[
        pltpu.VMEM((NSB, BLK), jnp.int32),
        pltpu.VMEM((NSB, BLK), jnp.int32),
        pltpu.VMEM((NSB, BLK), jnp.int32),
        pltpu.VMEM((2, BLK, D), jnp.float32),
        pltpu.VMEM((2, BLK), jnp.float32),
        pltpu.VMEM((2, BLK), jnp.float32),
        pltpu.VMEM((BLK,), jnp.float32),
        pltpu.VMEM((DZCH,), jnp.float32),
        pltpu.VMEM_SHARED((DEGSZ,), jnp.float32),
        pltpu.VMEM_SHARED((N, D), jnp.float32),
        pltpu.SemaphoreType.DMA,
        pltpu.SemaphoreType.DMA,
        pltpu.SemaphoreType.DMA,
        pltpu.SemaphoreType.DMA,
        pltpu.SemaphoreType.DMA,
        pltpu.SemaphoreType.DMA,
    ],
)
def _sc_edges(didx_hbm, gidx_hbm, tgt_hbm, h_hbm, outp_hbm,
              didx_v, gidx_v, tgt_v, rows_v,
              degv_v, scale_v, ones_v, zeros_v, deg_sp, out_sp,
              sem_g0, sem_g1, sem_d0, sem_d1, sem_s0, sem_s1):
    s = lax.axis_index("s")

    @pl.loop(0, BLK, step=16)
    def _(i):
        ones_v[pl.ds(i, 16)] = jnp.full((16,), 1.0, jnp.float32)

    @pl.loop(0, DZCH, step=16)
    def _(i):
        zeros_v[pl.ds(i, 16)] = jnp.full((16,), 0.0, jnp.float32)

    @pl.loop(s, NDZC, step=NS)
    def _(k):
        pltpu.sync_copy(zeros_v, deg_sp.at[pl.ds(k * DZCH, DZCH)])

    @pl.loop(0, BLK)
    def _(i):
        for f in range(D // 16):
            rows_v[0, i, pl.ds(f * 16, 16)] = (
                jnp.full((16,), 0.0, jnp.float32))

    @pl.loop(s, NZC, step=NS)
    def _(k):
        pltpu.sync_copy(rows_v.at[0], out_sp.at[pl.ds(k * ZCH, ZCH)])

    plsc.subcore_barrier()

    def _deg_group(cnt):
        descs = [pltpu.async_copy(ones_v, deg_sp.at[didx_v.at[j]],
                                  sem_d0, add=True) for j in range(cnt)]
        for d in descs:
            d.wait()

    @pl.loop(0, NGRP)
    def _(g):
        pltpu.sync_copy(didx_hbm.at[s].at[pl.ds(g * NSB, NSB)], didx_v)
        _deg_group(NSB)

    pltpu.sync_copy(didx_hbm.at[s].at[pl.ds(NGRP * NSB, TAIL)],
                    didx_v.at[pl.ds(0, TAIL)])
    _deg_group(TAIL)

    plsc.subcore_barrier()

    sem_g = (sem_g0, sem_g1)
    sem_d = (sem_d0, sem_d1)
    sem_s = (sem_s0, sem_s1)

    def _issue(j, p):
        dg = pltpu.async_copy(deg_sp.at[didx_v.at[j]], degv_v.at[p],
                              sem_d[p])
        rg = pltpu.async_copy(h_hbm.at[gidx_v.at[j]], rows_v.at[p],
                              sem_g[p])
        return dg, rg

    def _compute(p):
        @pl.loop(0, BLK, step=16)
        def _(i):
            scale_v[p, pl.ds(i, 16)] = (
                1.0 / jnp.maximum(degv_v[p, pl.ds(i, 16)], 1.0))

        @pl.loop(0, BLK, unroll=4)
        def _(i):
            sv = plsc.load_gather(
                scale_v,
                [jnp.full((16,), p, jnp.int32),
                 lax.broadcast_in_dim(i, (16,), ())])
            for f in range(D // 16):
                rows_v[p, i, pl.ds(f * 16, 16)] = (
                    rows_v[p, i, pl.ds(f * 16, 16)] * sv)

    def _pipeline(js):
        gath = [None, None]
        scat = [None, None]
        gath[0] = _issue(js[0], 0)
        for k, j in enumerate(js):
            p = k & 1
            q = p ^ 1
            if k + 1 < len(js):
                if scat[q] is not None:
                    scat[q].wait()
                    scat[q] = None
                gath[q] = _issue(js[k + 1], q)
            dg, rg = gath[p]
            dg.wait()
            rg.wait()
            _compute(p)
            scat[p] = pltpu.async_copy(rows_v.at[p],
                                       out_sp.at[tgt_v.at[j]], sem_s[p],
                                       add=True)
        for p in (0, 1):
            if scat[p] is not None:
                scat[p].wait()

    @pl.loop(0, NGRP)
    def _(g):
        pltpu.sync_copy(didx_hbm.at[s].at[pl.ds(g * NSB, NSB)], didx_v)
        pltpu.sync_copy(gidx_hbm.at[s].at[pl.ds(g * NSB, NSB)], gidx_v)
        pltpu.sync_copy(tgt_hbm.at[s].at[pl.ds(g * NSB, NSB)], tgt_v)
        _pipeline(list(range(NSB)))

    pltpu.sync_copy(didx_hbm.at[s].at[pl.ds(NGRP * NSB, TAIL)],
                    didx_v.at[pl.ds(0, TAIL)])
    pltpu.sync_copy(gidx_hbm.at[s].at[pl.ds(NGRP * NSB, TAIL)],
                    gidx_v.at[pl.ds(0, TAIL)])
    pltpu.sync_copy(tgt_hbm.at[s].at[pl.ds(NGRP * NSB, TAIL)],
                    tgt_v.at[pl.ds(0, TAIL)])
    _pipeline(list(range(TAIL)))

    plsc.subcore_barrier()

    @pl.loop(s, NZC, step=NS)
    def _(k):
        pltpu.sync_copy(out_sp.at[pl.ds(k * ZCH, ZCH)],
                        outp_hbm.at[pl.ds(k * ZCH, ZCH)])


def _comb_body(p_ref, r_ref, b_ref, o_ref):
    o_ref[...] = p_ref[...] + r_ref[...] + b_ref[...]


def _combine(outp, h, bias2d):
    nt = N // MB
    return pl.pallas_call(
        _comb_body,
        grid=(nt,),
        in_specs=[
            pl.BlockSpec((MB, D), lambda i: (i, 0)),
            pl.BlockSpec((MB, D), lambda i: (R * nt + i, 0)),
            pl.BlockSpec((1, D), lambda i: (0, 0)),
        ],
        out_specs=pl.BlockSpec((MB, D), lambda i: (i, 0)),
        out_shape=jax.ShapeDtypeStruct((N, D), jnp.float32),
    )(outp, h, bias2d)


def kernel(x, W, root_weight, bias, edge_index, edge_type):
    x = x.astype(jnp.float32)
    src = edge_index[0].astype(jnp.int32)
    tgt = edge_index[1].astype(jnp.int32)
    et = edge_type.astype(jnp.int32)

    wall = jnp.concatenate([W.astype(jnp.float32),
                            root_weight.astype(jnp.float32)[None]], axis=0)
    h = _relation_matmuls(x, wall)

    gidx = (et * N + src).reshape(NS, TPB, BLK)
    didx = (tgt * R + et).reshape(NS, TPB, BLK)
    tgt_r = tgt.reshape(NS, TPB, BLK)

    outp = _sc_edges(didx, gidx, tgt_r, h)

    return _combine(outp, h, bias.astype(jnp.float32).reshape(1, D))
